# baseline (speedup 1.0000x reference)
import functools

import jax
import jax.numpy as jnp
from jax import lax
from jax.experimental import pallas as pl
from jax.experimental.pallas import tpu as pltpu
from jax.experimental.pallas import tpu_sc as plsc

NC = 2
NS = 16
NW = NC * NS

G = 128
B = 4096 * 200
D = 128
NBUF = 5
LOOKAHEAD = 2

CPW = 140
N_SC = NW * G * CPW
N_TC = B - N_SC

M = 1024
VPAD = 1024


def _emb_lookup_sc(table, idx2d):
    mesh = plsc.VectorSubcoreMesh(core_axis_name="c", subcore_axis_name="s")

    @functools.partial(
        pl.kernel,
        out_type=jax.ShapeDtypeStruct((B, D), jnp.float32),
        mesh=mesh,
        scratch_types=[
            pltpu.VMEM((CPW, G), jnp.int32),
            [pltpu.VMEM((G, D), jnp.float32)] * NBUF,
            [pltpu.SemaphoreType.DMA] * NBUF,
            [pltpu.SemaphoreType.DMA] * NBUF,
        ],
    )
    def k(table_hbm, idx_hbm, out_hbm, idx_v, rows, gsems, osems):
        wid = lax.axis_index("s") * NC + lax.axis_index("c")
        base = wid * CPW
        pltpu.sync_copy(idx_hbm.at[wid], idx_v)

        def gather(c, s):
            return pltpu.make_async_copy(
                table_hbm.at[idx_v.at[c]], rows[s], gsems[s])

        def outcopy(c, s):
            return pltpu.make_async_copy(
                rows[s], out_hbm.at[pl.ds((base + c) * G, G)], osems[s])

        for c0 in range(LOOKAHEAD):
            gather(c0, c0).start()

        def body(go, carry):
            for b in range(NBUF):
                c = go * NBUF + b
                gather(c, b).wait()
                outcopy(c, b).start()
                s2 = (b + LOOKAHEAD) % NBUF

                @pl.when(c >= NBUF - LOOKAHEAD)
                def _():
                    outcopy(c + LOOKAHEAD - NBUF, s2).wait()

                @pl.when(c + LOOKAHEAD < CPW)
                def _():
                    gather(c + LOOKAHEAD, s2).start()

            return carry

        lax.fori_loop(0, CPW // NBUF, body, 0)

        for c in range(CPW - (NBUF - LOOKAHEAD), CPW):
            outcopy(c, c % NBUF).wait()

    return k(table, idx2d)


def _emb_lookup_tc(table_pad, idx3d):
    def body(idx_ref, tab_ref, o_ref):
        ohT = jnp.equal(
            lax.broadcasted_iota(jnp.int32, (VPAD, M), 0),
            jnp.broadcast_to(idx_ref[0], (VPAD, M)),
        ).astype(jnp.float32)
        o_ref[...] = lax.dot_general(
            ohT, tab_ref[...], (((0,), (0,)), ((), ())),
            preferred_element_type=jnp.float32)

    return pl.pallas_call(
        body,
        grid=(N_TC // M,),
        in_specs=[
            pl.BlockSpec((1, 1, M), lambda i: (i, 0, 0)),
            pl.BlockSpec((VPAD, D), lambda i: (0, 0)),
        ],
        out_specs=pl.BlockSpec((M, D), lambda i: (i, 0)),
        out_shape=jax.ShapeDtypeStruct((N_TC, D), jnp.float32),
    )(idx3d, table_pad)


def kernel(x, table):
    flat = x.reshape(-1).astype(jnp.int32)
    idx_sc = flat[:N_SC].reshape(NW, CPW, G)
    idx_tc = flat[N_SC:].reshape(-1, 1, M)
    table_pad = jnp.zeros((VPAD, D), jnp.float32).at[: table.shape[0]].set(table)

    big = _emb_lookup_sc(table, idx_sc)
    tc = _emb_lookup_tc(table_pad, idx_tc)
    out = lax.dynamic_update_slice(big, tc, (N_SC, 0))
    return out.reshape(x.shape + (D,))


# speedup vs baseline: 1.0140x; 1.0140x over previous
import functools

import jax
import jax.numpy as jnp
from jax import lax
from jax.experimental import pallas as pl
from jax.experimental.pallas import tpu as pltpu
from jax.experimental.pallas import tpu_sc as plsc

NC = 2
NS = 16
NW = NC * NS

G = 128
B = 4096 * 200
D = 128
NBUF = 5
LOOKAHEAD = 2

CPW = 150
N_SC = NW * G * CPW
N_TC = B - N_SC

M = 1024
VPAD = 1024


def _emb_lookup_sc(table, idx2d):
    mesh = plsc.VectorSubcoreMesh(core_axis_name="c", subcore_axis_name="s")

    @functools.partial(
        pl.kernel,
        out_type=jax.ShapeDtypeStruct((B, D), jnp.float32),
        mesh=mesh,
        scratch_types=[
            pltpu.VMEM((CPW, G), jnp.int32),
            [pltpu.VMEM((G, D), jnp.float32)] * NBUF,
            [pltpu.SemaphoreType.DMA] * NBUF,
            [pltpu.SemaphoreType.DMA] * NBUF,
        ],
    )
    def k(table_hbm, idx_hbm, out_hbm, idx_v, rows, gsems, osems):
        wid = lax.axis_index("s") * NC + lax.axis_index("c")
        base = wid * CPW
        pltpu.sync_copy(idx_hbm.at[wid], idx_v)

        def gather(c, s):
            return pltpu.make_async_copy(
                table_hbm.at[idx_v.at[c]], rows[s], gsems[s])

        def outcopy(c, s):
            return pltpu.make_async_copy(
                rows[s], out_hbm.at[pl.ds((base + c) * G, G)], osems[s])

        for c0 in range(LOOKAHEAD):
            gather(c0, c0).start()

        def body(go, carry):
            for b in range(NBUF):
                c = go * NBUF + b
                gather(c, b).wait()
                outcopy(c, b).start()
                s2 = (b + LOOKAHEAD) % NBUF

                @pl.when(c >= NBUF - LOOKAHEAD)
                def _():
                    outcopy(c + LOOKAHEAD - NBUF, s2).wait()

                @pl.when(c + LOOKAHEAD < CPW)
                def _():
                    gather(c + LOOKAHEAD, s2).start()

            return carry

        lax.fori_loop(0, CPW // NBUF, body, 0)

        for c in range(CPW - (NBUF - LOOKAHEAD), CPW):
            outcopy(c, c % NBUF).wait()

    return k(table, idx2d)


def _emb_lookup_tc(table_pad, idx3d):
    def body(idx_ref, tab_ref, o_ref):
        ohT = jnp.equal(
            lax.broadcasted_iota(jnp.int32, (VPAD, M), 0),
            jnp.broadcast_to(idx_ref[0], (VPAD, M)),
        ).astype(jnp.float32)
        o_ref[...] = lax.dot_general(
            ohT, tab_ref[...], (((0,), (0,)), ((), ())),
            preferred_element_type=jnp.float32)

    return pl.pallas_call(
        body,
        grid=(N_TC // M,),
        in_specs=[
            pl.BlockSpec((1, 1, M), lambda i: (i, 0, 0)),
            pl.BlockSpec((VPAD, D), lambda i: (0, 0)),
        ],
        out_specs=pl.BlockSpec((M, D), lambda i: (i, 0)),
        out_shape=jax.ShapeDtypeStruct((N_TC, D), jnp.float32),
    )(idx3d, table_pad)


def kernel(x, table):
    flat = x.reshape(-1).astype(jnp.int32)
    idx_sc = flat[:N_SC].reshape(NW, CPW, G)
    idx_tc = flat[N_SC:].reshape(-1, 1, M)
    table_pad = jnp.zeros((VPAD, D), jnp.float32).at[: table.shape[0]].set(table)

    big = _emb_lookup_sc(table, idx_sc)
    tc = _emb_lookup_tc(table_pad, idx_tc)
    out = lax.dynamic_update_slice(big, tc, (N_SC, 0))
    return out.reshape(x.shape + (D,))
